# baseline (speedup 1.0000x reference)
import functools

import jax
import jax.numpy as jnp
from jax.experimental import pallas as pl
from jax.experimental.pallas import tpu as pltpu

_N = 10000
_E = 320000
_D = 64
_B = 4
_T = 16
_BN = _B * _N
_BLK = 2000


def _dense_body(h_ref, s_ref, wh_ref, w4_ref, wo_ref, hn_ref, v_ref, vn_ref):
    pre = jnp.dot(h_ref[...], wh_ref[...], preferred_element_type=jnp.float32)
    pre = pre + jnp.dot(s_ref[...], w4_ref[...], preferred_element_type=jnp.float32)
    hn = jnp.tanh(pre)
    hn_ref[...] = hn
    vq = jnp.dot(hn, wo_ref[...], preferred_element_type=jnp.float32)
    v = jnp.clip(vq * 75.0 - 25.0, -100.0, 50.0)
    v_ref[...] = v
    vn_ref[...] = (v + 25.0) / 75.0


_dense_step = pl.pallas_call(
    _dense_body,
    grid=(_BN // _BLK,),
    in_specs=[
        pl.BlockSpec((_BLK, _D), lambda i: (i, 0)),
        pl.BlockSpec((_BLK, 4), lambda i: (i, 0)),
        pl.BlockSpec((_D, _D), lambda i: (0, 0)),
        pl.BlockSpec((4, _D), lambda i: (0, 0)),
        pl.BlockSpec((_D, 8), lambda i: (0, 0)),
    ],
    out_specs=[
        pl.BlockSpec((_BLK, _D), lambda i: (i, 0)),
        pl.BlockSpec((_BLK, 8), lambda i: (i, 0)),
        pl.BlockSpec((_BLK, 8), lambda i: (i, 0)),
    ],
    out_shape=[
        jax.ShapeDtypeStruct((_BN, _D), jnp.float32),
        jax.ShapeDtypeStruct((_BN, 8), jnp.float32),
        jax.ShapeDtypeStruct((_BN, 8), jnp.float32),
    ],
)


def kernel(input_traces, edge_index, meta_weight, w_msg, w_inp, W_h, b, w_out):
    src = edge_index[0]
    dst = edge_index[1]

    w4 = jnp.stack([w_msg, w_inp, b, jnp.zeros_like(b)], axis=0)
    wo = jnp.zeros((_D, 8), jnp.float32).at[:, 0].set(w_out)

    xs = jnp.moveaxis((input_traces + 25.0) / 75.0, -1, 0).reshape(_T, _BN)

    h0 = jnp.zeros((_BN, _D), jnp.float32)
    vn0 = jnp.full((_BN,), (-65.0 + 25.0) / 75.0, jnp.float32)
    ones = jnp.ones((_BN,), jnp.float32)
    zeros = jnp.zeros((_BN,), jnp.float32)

    def step(carry, x_t):
        h, vn = carry
        vn2 = vn.reshape(_B, _N)
        msg_e = vn2[:, src] * meta_weight[None, :]
        msg = jax.ops.segment_sum(msg_e.T, dst, num_segments=_N).T
        s = jnp.stack([msg.reshape(_BN), x_t, ones, zeros], axis=1)
        hn, v8, vn8 = _dense_step(h, s, W_h, w4, wo)
        return (hn, vn8[:, 0]), v8[:, 0]

    (_, _), vs = jax.lax.scan(step, (h0, vn0), xs)
    return jnp.moveaxis(vs.reshape(_T, _B, _N), 0, -1)


# speedup vs baseline: 11.8904x; 11.8904x over previous
import functools

import jax
import jax.numpy as jnp
from jax import lax
from jax.experimental import pallas as pl
from jax.experimental.pallas import tpu as pltpu
from jax.experimental.pallas import tpu_sc as plsc

_N = 10000
_E = 320000
_D = 64
_B = 4
_T = 16
_BN = _B * _N
_BLK = 2000
_NW = 32
_CH = _E // _NW


_sc_mesh = plsc.VectorSubcoreMesh(core_axis_name="c", subcore_axis_name="s")


@functools.partial(
    pl.kernel,
    mesh=_sc_mesh,
    compiler_params=pltpu.CompilerParams(needs_layout_passes=False),
    out_type=jax.ShapeDtypeStruct((2, _B, _N), jnp.float32),
    scratch_types=[
        pltpu.VMEM((_BN,), jnp.float32),
        pltpu.VMEM((_CH,), jnp.int32),
        pltpu.VMEM((_CH,), jnp.int32),
        pltpu.VMEM((_CH,), jnp.float32),
        pltpu.VMEM((_CH,), jnp.float32),
        pltpu.VMEM_SHARED((_N,), jnp.float32),
        pltpu.VMEM_SHARED((_N,), jnp.float32),
        pltpu.VMEM_SHARED((_N,), jnp.float32),
        pltpu.VMEM_SHARED((_N,), jnp.float32),
    ],
)
def _sc_msg(vn_hbm, src_hbm, dst_hbm, w_hbm, out_hbm,
            vn_v, src_v, dst_v, w_v, prod_v, m0, m1, m2, m3):
    cid = lax.axis_index("c")
    sid = lax.axis_index("s")
    gwid = cid * 16 + sid
    msgs = [m0, m1, m2, m3]

    @pl.when(sid < _B)
    def _zero():
        def zbody(i, c):
            prod_v[pl.ds(i * 16, 16)] = jnp.zeros((16,), jnp.float32)
            return c
        lax.fori_loop(0, _N // 16, zbody, 0)
        for b in range(_B):
            @pl.when(sid == b)
            def _cp(b=b):
                pltpu.sync_copy(prod_v, msgs[b])

    pltpu.sync_copy(vn_hbm, vn_v)
    pltpu.sync_copy(src_hbm.at[gwid], src_v)
    pltpu.sync_copy(dst_hbm.at[gwid], dst_v)
    pltpu.sync_copy(w_hbm.at[gwid], w_v)

    plsc.subcore_barrier()

    for b in range(_B):
        boff = jnp.full((16,), b * _N, jnp.int32)

        def gbody(i, c, boff=boff):
            s16 = src_v[pl.ds(i * 16, 16)]
            vals = plsc.load_gather(vn_v, [s16 + boff])
            w16 = w_v[pl.ds(i * 16, 16)]
            prod_v[pl.ds(i * 16, 16)] = vals * w16
            return c
        lax.fori_loop(0, _CH // 16, gbody, 0)
        pltpu.sync_copy(prod_v, msgs[b].at[dst_v], add=True)

    plsc.subcore_barrier()

    @pl.when(sid < _B)
    def _writeout():
        for b in range(_B):
            @pl.when(sid == b)
            def _cp(b=b):
                pltpu.sync_copy(msgs[b], out_hbm.at[cid, b])


def _dense_body(h_ref, s_ref, wh_ref, w4_ref, wo_ref, hn_ref, v_ref, vn_ref):
    pre = jnp.dot(h_ref[...], wh_ref[...], preferred_element_type=jnp.float32)
    pre = pre + jnp.dot(s_ref[...], w4_ref[...],
                        preferred_element_type=jnp.float32,
                        precision=lax.Precision.HIGHEST)
    hn = jnp.tanh(pre)
    hn_ref[...] = hn
    vq = jnp.dot(hn, wo_ref[...], preferred_element_type=jnp.float32,
                 precision=lax.Precision.HIGHEST)
    v = jnp.clip(vq * 75.0 - 25.0, -100.0, 50.0)
    v_ref[...] = v
    vn_ref[...] = (v + 25.0) / 75.0


_dense_step = pl.pallas_call(
    _dense_body,
    grid=(_BN // _BLK,),
    in_specs=[
        pl.BlockSpec((_BLK, _D), lambda i: (i, 0)),
        pl.BlockSpec((_BLK, 4), lambda i: (i, 0)),
        pl.BlockSpec((_D, _D), lambda i: (0, 0)),
        pl.BlockSpec((4, _D), lambda i: (0, 0)),
        pl.BlockSpec((_D, 8), lambda i: (0, 0)),
    ],
    out_specs=[
        pl.BlockSpec((_BLK, _D), lambda i: (i, 0)),
        pl.BlockSpec((_BLK, 8), lambda i: (i, 0)),
        pl.BlockSpec((_BLK, 8), lambda i: (i, 0)),
    ],
    out_shape=[
        jax.ShapeDtypeStruct((_BN, _D), jnp.float32),
        jax.ShapeDtypeStruct((_BN, 8), jnp.float32),
        jax.ShapeDtypeStruct((_BN, 8), jnp.float32),
    ],
)


def kernel(input_traces, edge_index, meta_weight, w_msg, w_inp, W_h, b, w_out):
    src2 = edge_index[0].reshape(_NW, _CH)
    dst2 = edge_index[1].reshape(_NW, _CH)
    w2 = meta_weight.reshape(_NW, _CH)

    w4 = jnp.stack([w_msg, w_msg, w_inp, b], axis=0)
    wo = jnp.zeros((_D, 8), jnp.float32).at[:, 0].set(w_out)

    xs = jnp.moveaxis((input_traces + 25.0) / 75.0, -1, 0).reshape(_T, _BN)

    h0 = jnp.zeros((_BN, _D), jnp.float32)
    vn0 = jnp.full((_BN,), (-65.0 + 25.0) / 75.0, jnp.float32)
    ones = jnp.ones((_BN,), jnp.float32)

    def step(carry, x_t):
        h, vn = carry
        mp = _sc_msg(vn, src2, dst2, w2)
        s = jnp.stack(
            [mp[0].reshape(_BN), mp[1].reshape(_BN), x_t, ones], axis=1)
        hn, v8, vn8 = _dense_step(h, s, W_h, w4, wo)
        return (hn, vn8[:, 0]), v8[:, 0]

    (_, _), vs = jax.lax.scan(step, (h0, vn0), xs)
    return jnp.moveaxis(vs.reshape(_T, _B, _N), 0, -1)


# speedup vs baseline: 16.6293x; 1.3985x over previous
import functools

import jax
import jax.numpy as jnp
from jax import lax
from jax.experimental import pallas as pl
from jax.experimental.pallas import tpu as pltpu
from jax.experimental.pallas import tpu_sc as plsc

_N = 10000
_E = 320000
_D = 64
_B = 4
_T = 16
_BN = _B * _N
_BLK = 2000
_NW = 32
_CH = _E // _NW


_sc_mesh = plsc.VectorSubcoreMesh(core_axis_name="c", subcore_axis_name="s")


@functools.partial(
    pl.kernel,
    mesh=_sc_mesh,
    compiler_params=pltpu.CompilerParams(needs_layout_passes=False),
    out_type=jax.ShapeDtypeStruct((2, _B, _N), jnp.float32),
    scratch_types=[
        pltpu.VMEM((_BN,), jnp.float32),
        pltpu.VMEM((_CH,), jnp.int32),
        pltpu.VMEM((_CH,), jnp.int32),
        pltpu.VMEM((_CH,), jnp.float32),
        pltpu.VMEM((_CH,), jnp.float32),
        pltpu.VMEM((_CH,), jnp.float32),
        pltpu.VMEM_SHARED((_N,), jnp.float32),
        pltpu.VMEM_SHARED((_N,), jnp.float32),
        pltpu.VMEM_SHARED((_N,), jnp.float32),
        pltpu.VMEM_SHARED((_N,), jnp.float32),
        pltpu.SemaphoreType.DMA,
    ],
)
def _sc_msg(vn_hbm, src_hbm, dst_hbm, w_hbm, out_hbm,
            vn_v, src_v, dst_v, w_v, prod_a, prod_b, m0, m1, m2, m3, sem):
    cid = lax.axis_index("c")
    sid = lax.axis_index("s")
    gwid = cid * 16 + sid
    msgs = [m0, m1, m2, m3]

    c_vn = pltpu.async_copy(vn_hbm, vn_v, sem)
    c_src = pltpu.async_copy(src_hbm.at[gwid], src_v, sem)
    c_dst = pltpu.async_copy(dst_hbm.at[gwid], dst_v, sem)
    c_w = pltpu.async_copy(w_hbm.at[gwid], w_v, sem)

    @pl.when(sid < _B)
    def _zero():
        @plsc.parallel_loop(0, _N, 16, unroll=4)
        def _zbody(i):
            prod_b[pl.ds(i, 16)] = jnp.zeros((16,), jnp.float32)
        for b in range(_B):
            @pl.when(sid == b)
            def _cp(b=b):
                pltpu.sync_copy(prod_b, msgs[b])

    c_vn.wait()
    c_src.wait()
    c_dst.wait()
    c_w.wait()

    plsc.subcore_barrier()

    prods = [prod_a, prod_b]
    copies = [None] * _B
    for b in range(_B):
        boff = jnp.full((16,), b * _N, jnp.int32)
        pv = prods[b % 2]
        if b >= 2:
            copies[b - 2].wait()

        @plsc.parallel_loop(0, _CH, 16, unroll=4)
        def _gbody(i, boff=boff, pv=pv):
            s16 = src_v[pl.ds(i, 16)]
            vals = plsc.load_gather(vn_v, [s16 + boff])
            w16 = w_v[pl.ds(i, 16)]
            pv[pl.ds(i, 16)] = vals * w16
        copies[b] = pltpu.async_copy(pv, msgs[b].at[dst_v], sem, add=True)

    copies[_B - 2].wait()
    copies[_B - 1].wait()
    plsc.subcore_barrier()

    @pl.when(sid < _B)
    def _writeout():
        for b in range(_B):
            @pl.when(sid == b)
            def _cp(b=b):
                pltpu.sync_copy(msgs[b], out_hbm.at[cid, b])


def _dense_body(h_ref, s_ref, wh_ref, w4_ref, wo_ref, hn_ref, v_ref, vn_ref):
    s = s_ref[...]
    msgb = s[:, 0:1] + s[:, 1:2]
    xb = s[:, 2:3]
    pre = msgb * w4_ref[0:1, :] + xb * w4_ref[2:3, :]
    pre = pre + jnp.dot(h_ref[...], wh_ref[...], preferred_element_type=jnp.float32)
    pre = pre + w4_ref[3:4, :]
    hn = jnp.tanh(pre)
    hn_ref[...] = hn
    vq = jnp.dot(hn, wo_ref[...].T, preferred_element_type=jnp.float32)
    v = jnp.clip(vq * 75.0 - 25.0, -100.0, 50.0)
    v_ref[...] = v
    vn_ref[...] = (v + 25.0) / 75.0


_dense_step = pl.pallas_call(
    _dense_body,
    grid=(_BN // _BLK,),
    in_specs=[
        pl.BlockSpec((_BLK, _D), lambda i: (i, 0)),
        pl.BlockSpec((_BLK, 4), lambda i: (i, 0)),
        pl.BlockSpec((_D, _D), lambda i: (0, 0)),
        pl.BlockSpec((4, _D), lambda i: (0, 0)),
        pl.BlockSpec((8, _D), lambda i: (0, 0)),
    ],
    out_specs=[
        pl.BlockSpec((_BLK, _D), lambda i: (i, 0)),
        pl.BlockSpec((_BLK, 8), lambda i: (i, 0)),
        pl.BlockSpec((_BLK, 8), lambda i: (i, 0)),
    ],
    out_shape=[
        jax.ShapeDtypeStruct((_BN, _D), jnp.float32),
        jax.ShapeDtypeStruct((_BN, 8), jnp.float32),
        jax.ShapeDtypeStruct((_BN, 8), jnp.float32),
    ],
)


def kernel(input_traces, edge_index, meta_weight, w_msg, w_inp, W_h, b, w_out):
    src2 = edge_index[0].reshape(_NW, _CH)
    dst2 = edge_index[1].reshape(_NW, _CH)
    w2 = meta_weight.reshape(_NW, _CH)

    w4 = jnp.stack([w_msg, w_msg, w_inp, b], axis=0)
    wo = jnp.zeros((8, _D), jnp.float32).at[0, :].set(w_out)

    xs = jnp.moveaxis((input_traces + 25.0) / 75.0, -1, 0).reshape(_T, _BN)

    h0 = jnp.zeros((_BN, _D), jnp.float32)
    vn0 = jnp.full((_BN,), (-65.0 + 25.0) / 75.0, jnp.float32)
    ones = jnp.ones((_BN,), jnp.float32)

    def step(carry, x_t):
        h, vn = carry
        mp = _sc_msg(vn, src2, dst2, w2)
        s = jnp.stack(
            [mp[0].reshape(_BN), mp[1].reshape(_BN), x_t, ones], axis=1)
        hn, v8, vn8 = _dense_step(h, s, W_h, w4, wo)
        return (hn, vn8[:, 0]), v8[:, 0]

    (_, _), vs = jax.lax.scan(step, (h0, vn0), xs)
    return jnp.moveaxis(vs.reshape(_T, _B, _N), 0, -1)
